# baseline (speedup 1.0000x reference)
import functools

import jax
import jax.numpy as jnp
from jax import lax
from jax.experimental import pallas as pl
from jax.experimental.pallas import tpu as pltpu
from jax.experimental.pallas import tpu_sc as plsc

N_NODES = 10000
NP = 10240
N_EDGES = 320000
NCORE = 2
NSUB = 16
NW = NCORE * NSUB
CHUNK = 112
TSTEPS = 180
S0 = 156
S1 = TSTEPS - S0
DSTEPS = TSTEPS // 2
EPT = DSTEPS * CHUNK
EPAD = NW * EPT
PAD_NODE = NP - 1
NBUF = 3
RPT = NP // NSUB

_MESH = plsc.VectorSubcoreMesh(core_axis_name="c", subcore_axis_name="s")


@functools.partial(
    pl.kernel,
    out_type=jax.ShapeDtypeStruct((NCORE, 1, NP), jnp.float32),
    mesh=_MESH,
    scratch_types=[
        pltpu.VMEM((DSTEPS, CHUNK), jnp.int32),
        pltpu.VMEM((CHUNK,), jnp.float32),
        pltpu.VMEM_SHARED((NP,), jnp.float32),
        pltpu.SemaphoreType.DMA,
    ],
)
def _sc_degree(colr, zeros, out, colbuf, ones, acc, sem):
    cid = lax.axis_index("c")
    sid = lax.axis_index("s")
    wid = cid * NSUB + sid
    for j in range(CHUNK // 16):
        ones[pl.ds(j * 16, 16)] = jnp.ones((16,), jnp.float32)
    b0 = pl.multiple_of(sid * RPT, 128)
    pltpu.sync_copy(zeros.at[pl.ds(b0, RPT)], acc.at[pl.ds(b0, RPT)])
    pltpu.sync_copy(colr.at[wid], colbuf)
    plsc.subcore_barrier()

    def fire(i, carry):
        pltpu.async_copy(ones, acc.at[colbuf.at[i]], sem, add=True)
        return carry

    lax.fori_loop(0, DSTEPS, fire, 0)

    def drain(i, carry):
        pltpu.make_async_copy(ones, acc.at[colbuf.at[0]], sem).wait()
        return carry

    lax.fori_loop(0, DSTEPS, drain, 0)
    plsc.subcore_barrier()
    pltpu.sync_copy(acc.at[pl.ds(b0, RPT)], out.at[cid, 0, pl.ds(b0, RPT)])


@functools.cache
def _sc_gather_scatter(feat):
    @functools.partial(
        pl.kernel,
        out_type=jax.ShapeDtypeStruct((NCORE, NP, feat), jnp.float32),
        mesh=_MESH,
        scratch_types=[pltpu.VMEM((2, CHUNK), jnp.int32)] * NBUF
          + [pltpu.VMEM((CHUNK, feat), jnp.float32)] * NBUF
          + [pltpu.VMEM_SHARED((NP, feat), jnp.float32)]
          + [pltpu.SemaphoreType.DMA] * NBUF,
    )
    def body(src, rc, out, *refs):
        rcb = refs[0:NBUF]
        msg = refs[NBUF:2 * NBUF]
        acc = refs[2 * NBUF]
        sem = refs[2 * NBUF + 1:3 * NBUF + 1]
        cid = lax.axis_index("c")
        sid = lax.axis_index("s")
        r0 = sid * RPT
        base = jnp.where(cid == 0, 0, S0)
        nsteps = jnp.where(cid == 0, S0, S1)

        pltpu.sync_copy(src.at[pl.ds(r0, RPT)], acc.at[pl.ds(r0, RPT)])
        plsc.subcore_barrier()

        for b in range(NBUF):
            pltpu.sync_copy(rc.at[sid, base + b], rcb[b])
            pltpu.async_copy(src.at[rcb[b].at[0]], msg[b], sem[b])

        def group(g, carry):
            for b in range(NBUF):
                i = g * NBUF + b
                pltpu.make_async_copy(src.at[rcb[b].at[0]], msg[b],
                                      sem[b]).wait()
                pltpu.sync_copy(msg[b], acc.at[rcb[b].at[1]], add=True)
                nxt = i + NBUF

                @pl.when(nxt < nsteps)
                def _prefetch():
                    pltpu.sync_copy(rc.at[sid, base + nxt], rcb[b])
                    pltpu.async_copy(src.at[rcb[b].at[0]], msg[b], sem[b])
            return carry

        lax.fori_loop(0, nsteps // NBUF, group, 0)
        plsc.subcore_barrier()
        pltpu.sync_copy(acc.at[pl.ds(r0, RPT)], out.at[cid, pl.ds(r0, RPT)])

    return body


_R = 1024
_GRID = NP // _R


def _row_spec(*block):
    return pl.BlockSpec(block, lambda i: (i,) + (0,) * (len(block) - 1))


def _const_spec(*block):
    return pl.BlockSpec(block, lambda i: (0,) * len(block))


def _part_spec(*block):
    return pl.BlockSpec(block, lambda i: (0, i) + (0,) * (len(block) - 2))


def _t1_body(x_ref, d_ref, w_ref, xs_ref, y0_ref):
    x = x_ref[...]
    xs_ref[...] = x * d_ref[...]
    y0_ref[...] = jnp.dot(x, w_ref[...], preferred_element_type=jnp.float32)


_t1 = pl.pallas_call(
    _t1_body,
    grid=(_GRID,),
    in_specs=[_row_spec(_R, 128), _row_spec(_R, 1), _const_spec(128, 128)],
    out_specs=[_row_spec(_R, 128), _row_spec(_R, 128)],
    out_shape=[jax.ShapeDtypeStruct((NP, 128), jnp.float32)] * 2,
)


def _t2_body(s_ref, xs_ref, d_ref, d2_ref, w_ref, y1_ref, t1_ref):
    u = s_ref[0] + s_ref[1] - xs_ref[...]
    y1_ref[...] = jnp.dot(u * d_ref[...], w_ref[...],
                          preferred_element_type=jnp.float32)
    t1_ref[...] = u * d2_ref[...]


_t2 = pl.pallas_call(
    _t2_body,
    grid=(_GRID,),
    in_specs=[_part_spec(2, _R, 128), _row_spec(_R, 128), _row_spec(_R, 1),
              _row_spec(_R, 1), _const_spec(128, 128)],
    out_specs=[_row_spec(_R, 128), _row_spec(_R, 128)],
    out_shape=[jax.ShapeDtypeStruct((NP, 128), jnp.float32)] * 2,
)


def _t3_body(s_ref, t1_ref, y0_ref, y1_ref, d_ref, w0c_ref, b0a_ref, b0b_ref,
             b0c_ref, wa0_ref, wa1_ref, wa2_ref, wb0_ref, wb1_ref, wb2_ref,
             wc0_ref, wc1_ref, wc2_ref, ga_ref, gb_ref, gc_ref):
    d = d_ref[...]
    u2 = s_ref[0] + s_ref[1] - t1_ref[...]
    y2 = jnp.dot(u2 * d, w0c_ref[...], preferred_element_type=jnp.float32)
    h0 = jnp.maximum(y0_ref[...] + b0a_ref[...], 0.0)
    h1 = jnp.maximum(y1_ref[...] + b0b_ref[...], 0.0)
    h2 = jnp.maximum(y2 + b0c_ref[...], 0.0)

    def mm3(w0_ref, w1_ref, w2_ref):
        return (jnp.dot(h0, w0_ref[...], preferred_element_type=jnp.float32)
                + jnp.dot(h1, w1_ref[...], preferred_element_type=jnp.float32)
                + jnp.dot(h2, w2_ref[...], preferred_element_type=jnp.float32))

    ga_ref[...] = mm3(wa0_ref, wa1_ref, wa2_ref)
    gb_ref[...] = mm3(wb0_ref, wb1_ref, wb2_ref) * d
    gc_ref[...] = mm3(wc0_ref, wc1_ref, wc2_ref) * d


_t3 = pl.pallas_call(
    _t3_body,
    grid=(_GRID,),
    in_specs=[_part_spec(2, _R, 128), _row_spec(_R, 128), _row_spec(_R, 128),
              _row_spec(_R, 128), _row_spec(_R, 1), _const_spec(128, 128)]
             + [_const_spec(1, 128)] * 3 + [_const_spec(128, 40)] * 9,
    out_specs=[_row_spec(_R, 40)] * 3,
    out_shape=[jax.ShapeDtypeStruct((NP, 40), jnp.float32)] * 3,
)


def _t4_body(s_ref, g_ref, dd_ref, v_ref):
    v_ref[...] = (s_ref[0] + s_ref[1] - g_ref[...]) * dd_ref[...]


_t4 = pl.pallas_call(
    _t4_body,
    grid=(_GRID,),
    in_specs=[_part_spec(2, _R, 2, 64), _row_spec(_R, 2, 64),
              _row_spec(_R, 2, 1)],
    out_specs=[_row_spec(_R, 2, 64)],
    out_shape=[jax.ShapeDtypeStruct((NP, 2, 64), jnp.float32)],
)


def _t5_body(s_ref, t2_ref, ga_ref, p_ref, d_ref, b1a_ref, b1b_ref, b1c_ref,
             o0_ref, o1_ref, o2_ref):
    q = (s_ref[0][:, 0:40] + s_ref[1][:, 0:40] - t2_ref[...]) * d_ref[...]
    l0 = ga_ref[...] + b1a_ref[...]
    l1 = p_ref[...] + b1b_ref[...]
    l2 = q + b1c_ref[...]
    m = jnp.maximum(jnp.maximum(jnp.max(l0, axis=1, keepdims=True),
                                jnp.max(l1, axis=1, keepdims=True)),
                    jnp.max(l2, axis=1, keepdims=True))
    e0 = jnp.exp(l0 - m)
    e1 = jnp.exp(l1 - m)
    e2 = jnp.exp(l2 - m)
    tot = (jnp.sum(e0, axis=1, keepdims=True)
           + jnp.sum(e1, axis=1, keepdims=True)
           + jnp.sum(e2, axis=1, keepdims=True))
    lz = m + jnp.log(tot)
    o0_ref[...] = l0 - lz
    o1_ref[...] = l1 - lz
    o2_ref[...] = l2 - lz


_t5 = pl.pallas_call(
    _t5_body,
    grid=(_GRID,),
    in_specs=[_part_spec(2, _R, 128), _row_spec(_R, 40), _row_spec(_R, 40),
              _row_spec(_R, 40), _row_spec(_R, 1)] + [_const_spec(1, 40)] * 3,
    out_specs=[_row_spec(_R, 40)] * 3,
    out_shape=[jax.ShapeDtypeStruct((NP, 40), jnp.float32)] * 3,
)


def kernel(x, adj, W0a, W0b, W0c, b0, W1a, W1b, W1c, b1):
    epad = jnp.full((2, EPAD - N_EDGES), PAD_NODE, jnp.int32)
    adj32 = jnp.concatenate([adj.astype(jnp.int32), epad], axis=1)
    rcw = adj32.reshape(2, NSUB, TSTEPS, CHUNK)
    rc = jnp.stack([rcw[0], rcw[1]], axis=2)
    colr = adj32[1].reshape(NW, DSTEPS, CHUNK)
    xp = jnp.pad(x, ((0, NP - N_NODES), (0, 0)))
    z1 = jnp.zeros((NP,), jnp.float32)

    degp = _sc_degree(colr, z1)
    deg = degp[0, 0] + degp[1, 0] + 1.0
    d = lax.rsqrt(deg)[:, None]
    d2 = (1.0 / deg)[:, None]
    dd = jnp.stack([d, d2], axis=1)

    xs, y0 = _t1(xp, d, W0a)
    s1 = _sc_gather_scatter(128)(xs, rc)
    y1, t1v = _t2(s1, xs, d, d2, W0b)
    s2 = _sc_gather_scatter(128)(t1v, rc)

    ga, gb, gc = _t3(
        s2, t1v, y0, y1, d, W0c,
        b0[None, 0:128], b0[None, 128:256], b0[None, 256:384],
        W1a[0:128], W1a[128:256], W1a[256:384],
        W1b[0:128], W1b[128:256], W1b[256:384],
        W1c[0:128], W1c[128:256], W1c[256:384])
    z24 = jnp.zeros((NP, 24), jnp.float32)
    gbc = jnp.concatenate([gb, z24, gc, z24], axis=1)
    s3 = _sc_gather_scatter(128)(gbc, rc)
    (v,) = _t4(s3.reshape(NCORE, NP, 2, 64),
               gbc.reshape(NP, 2, 64), dd)
    p = v[:, 0, 0:40]
    t2v = jnp.pad(v[:, 1, :], ((0, 0), (0, 64)))
    s4 = _sc_gather_scatter(128)(t2v, rc)
    o0, o1, o2 = _t5(s4, v[:, 1, 0:40], ga, p, d,
                     b1[None, 0:40], b1[None, 40:80], b1[None, 80:120])
    return jnp.concatenate([o0, o1, o2], axis=1)[:N_NODES]


# speedup vs baseline: 1.0620x; 1.0620x over previous
import functools

import jax
import jax.numpy as jnp
from jax import lax
from jax.experimental import pallas as pl
from jax.experimental.pallas import tpu as pltpu
from jax.experimental.pallas import tpu_sc as plsc

N_NODES = 10000
NP = 10240
N_EDGES = 320000
NCORE = 2
NSUB = 16
NW = NCORE * NSUB
CHUNK = 112
TSTEPS = 180
S0 = 141
S1 = TSTEPS - S0
DSTEPS = TSTEPS // 2
EPT = DSTEPS * CHUNK
EPAD = NW * EPT
PAD_NODE = NP - 1
NBUF = 3
RPT = NP // NSUB

_MESH = plsc.VectorSubcoreMesh(core_axis_name="c", subcore_axis_name="s")


@functools.partial(
    pl.kernel,
    out_type=jax.ShapeDtypeStruct((NCORE, 1, NP), jnp.float32),
    mesh=_MESH,
    scratch_types=[
        pltpu.VMEM((DSTEPS, CHUNK), jnp.int32),
        pltpu.VMEM((CHUNK,), jnp.float32),
        pltpu.VMEM_SHARED((NP,), jnp.float32),
        pltpu.SemaphoreType.DMA,
    ],
)
def _sc_degree(colr, zeros, out, colbuf, ones, acc, sem):
    cid = lax.axis_index("c")
    sid = lax.axis_index("s")
    wid = cid * NSUB + sid
    for j in range(CHUNK // 16):
        ones[pl.ds(j * 16, 16)] = jnp.ones((16,), jnp.float32)
    b0 = pl.multiple_of(sid * RPT, 128)
    pltpu.sync_copy(zeros.at[pl.ds(b0, RPT)], acc.at[pl.ds(b0, RPT)])
    pltpu.sync_copy(colr.at[wid], colbuf)
    plsc.subcore_barrier()

    def fire(i, carry):
        pltpu.async_copy(ones, acc.at[colbuf.at[i]], sem, add=True)
        return carry

    lax.fori_loop(0, DSTEPS, fire, 0)

    def drain(i, carry):
        pltpu.make_async_copy(ones, acc.at[colbuf.at[0]], sem).wait()
        return carry

    lax.fori_loop(0, DSTEPS, drain, 0)
    plsc.subcore_barrier()
    pltpu.sync_copy(acc.at[pl.ds(b0, RPT)], out.at[cid, 0, pl.ds(b0, RPT)])


@functools.cache
def _sc_gather_scatter(feat):
    @functools.partial(
        pl.kernel,
        out_type=jax.ShapeDtypeStruct((NCORE, NP, feat), jnp.float32),
        mesh=_MESH,
        scratch_types=[pltpu.VMEM((2, CHUNK), jnp.int32)] * NBUF
          + [pltpu.VMEM((CHUNK, feat), jnp.float32)] * NBUF
          + [pltpu.VMEM_SHARED((NP, feat), jnp.float32)]
          + [pltpu.SemaphoreType.DMA] * NBUF,
    )
    def body(src, rc, out, *refs):
        rcb = refs[0:NBUF]
        msg = refs[NBUF:2 * NBUF]
        acc = refs[2 * NBUF]
        sem = refs[2 * NBUF + 1:3 * NBUF + 1]
        cid = lax.axis_index("c")
        sid = lax.axis_index("s")
        r0 = sid * RPT
        base = jnp.where(cid == 0, 0, S0)
        nsteps = jnp.where(cid == 0, S0, S1)

        pltpu.sync_copy(src.at[pl.ds(r0, RPT)], acc.at[pl.ds(r0, RPT)])
        plsc.subcore_barrier()

        for b in range(NBUF):
            pltpu.sync_copy(rc.at[sid, base + b], rcb[b])
            pltpu.async_copy(src.at[rcb[b].at[0]], msg[b], sem[b])

        def group(g, carry):
            for b in range(NBUF):
                i = g * NBUF + b
                pltpu.make_async_copy(src.at[rcb[b].at[0]], msg[b],
                                      sem[b]).wait()
                pltpu.sync_copy(msg[b], acc.at[rcb[b].at[1]], add=True)
                nxt = i + NBUF

                @pl.when(nxt < nsteps)
                def _prefetch():
                    pltpu.sync_copy(rc.at[sid, base + nxt], rcb[b])
                    pltpu.async_copy(src.at[rcb[b].at[0]], msg[b], sem[b])
            return carry

        lax.fori_loop(0, nsteps // NBUF, group, 0)
        plsc.subcore_barrier()
        pltpu.sync_copy(acc.at[pl.ds(r0, RPT)], out.at[cid, pl.ds(r0, RPT)])

    return body


_R = 1024
_GRID = NP // _R


def _row_spec(*block):
    return pl.BlockSpec(block, lambda i: (i,) + (0,) * (len(block) - 1))


def _const_spec(*block):
    return pl.BlockSpec(block, lambda i: (0,) * len(block))


def _part_spec(*block):
    return pl.BlockSpec(block, lambda i: (0, i) + (0,) * (len(block) - 2))


def _t1_body(x_ref, d_ref, w_ref, xs_ref, y0_ref):
    x = x_ref[...]
    xs_ref[...] = x * d_ref[...]
    y0_ref[...] = jnp.dot(x, w_ref[...], preferred_element_type=jnp.float32)


_t1 = pl.pallas_call(
    _t1_body,
    grid=(_GRID,),
    in_specs=[_row_spec(_R, 128), _row_spec(_R, 1), _const_spec(128, 128)],
    out_specs=[_row_spec(_R, 128), _row_spec(_R, 128)],
    out_shape=[jax.ShapeDtypeStruct((NP, 128), jnp.float32)] * 2,
)


def _t2_body(s_ref, xs_ref, d_ref, d2_ref, w_ref, y1_ref, t1_ref):
    u = s_ref[0] + s_ref[1] - xs_ref[...]
    y1_ref[...] = jnp.dot(u * d_ref[...], w_ref[...],
                          preferred_element_type=jnp.float32)
    t1_ref[...] = u * d2_ref[...]


_t2 = pl.pallas_call(
    _t2_body,
    grid=(_GRID,),
    in_specs=[_part_spec(2, _R, 128), _row_spec(_R, 128), _row_spec(_R, 1),
              _row_spec(_R, 1), _const_spec(128, 128)],
    out_specs=[_row_spec(_R, 128), _row_spec(_R, 128)],
    out_shape=[jax.ShapeDtypeStruct((NP, 128), jnp.float32)] * 2,
)


def _t3_body(s_ref, t1_ref, y0_ref, y1_ref, d_ref, w0c_ref, b0a_ref, b0b_ref,
             b0c_ref, wa0_ref, wa1_ref, wa2_ref, wb0_ref, wb1_ref, wb2_ref,
             wc0_ref, wc1_ref, wc2_ref, ga_ref, gb_ref, gc_ref):
    d = d_ref[...]
    u2 = s_ref[0] + s_ref[1] - t1_ref[...]
    y2 = jnp.dot(u2 * d, w0c_ref[...], preferred_element_type=jnp.float32)
    h0 = jnp.maximum(y0_ref[...] + b0a_ref[...], 0.0)
    h1 = jnp.maximum(y1_ref[...] + b0b_ref[...], 0.0)
    h2 = jnp.maximum(y2 + b0c_ref[...], 0.0)

    def mm3(w0_ref, w1_ref, w2_ref):
        return (jnp.dot(h0, w0_ref[...], preferred_element_type=jnp.float32)
                + jnp.dot(h1, w1_ref[...], preferred_element_type=jnp.float32)
                + jnp.dot(h2, w2_ref[...], preferred_element_type=jnp.float32))

    ga_ref[...] = mm3(wa0_ref, wa1_ref, wa2_ref)
    gb_ref[...] = mm3(wb0_ref, wb1_ref, wb2_ref) * d
    gc_ref[...] = mm3(wc0_ref, wc1_ref, wc2_ref) * d


_t3 = pl.pallas_call(
    _t3_body,
    grid=(_GRID,),
    in_specs=[_part_spec(2, _R, 128), _row_spec(_R, 128), _row_spec(_R, 128),
              _row_spec(_R, 128), _row_spec(_R, 1), _const_spec(128, 128)]
             + [_const_spec(1, 128)] * 3 + [_const_spec(128, 40)] * 9,
    out_specs=[_row_spec(_R, 40)] * 3,
    out_shape=[jax.ShapeDtypeStruct((NP, 40), jnp.float32)] * 3,
)


def _t4_body(s_ref, g_ref, dd_ref, v_ref):
    v_ref[...] = (s_ref[0] + s_ref[1] - g_ref[...]) * dd_ref[...]


_t4 = pl.pallas_call(
    _t4_body,
    grid=(_GRID,),
    in_specs=[_part_spec(2, _R, 2, 64), _row_spec(_R, 2, 64),
              _row_spec(_R, 2, 1)],
    out_specs=[_row_spec(_R, 2, 64)],
    out_shape=[jax.ShapeDtypeStruct((NP, 2, 64), jnp.float32)],
)


def _t5_body(s_ref, t2_ref, ga_ref, p_ref, d_ref, b1a_ref, b1b_ref, b1c_ref,
             o0_ref, o1_ref, o2_ref):
    q = (s_ref[0][:, 0:40] + s_ref[1][:, 0:40] - t2_ref[...]) * d_ref[...]
    l0 = ga_ref[...] + b1a_ref[...]
    l1 = p_ref[...] + b1b_ref[...]
    l2 = q + b1c_ref[...]
    m = jnp.maximum(jnp.maximum(jnp.max(l0, axis=1, keepdims=True),
                                jnp.max(l1, axis=1, keepdims=True)),
                    jnp.max(l2, axis=1, keepdims=True))
    e0 = jnp.exp(l0 - m)
    e1 = jnp.exp(l1 - m)
    e2 = jnp.exp(l2 - m)
    tot = (jnp.sum(e0, axis=1, keepdims=True)
           + jnp.sum(e1, axis=1, keepdims=True)
           + jnp.sum(e2, axis=1, keepdims=True))
    lz = m + jnp.log(tot)
    o0_ref[...] = l0 - lz
    o1_ref[...] = l1 - lz
    o2_ref[...] = l2 - lz


_t5 = pl.pallas_call(
    _t5_body,
    grid=(_GRID,),
    in_specs=[_part_spec(2, _R, 128), _row_spec(_R, 40), _row_spec(_R, 40),
              _row_spec(_R, 40), _row_spec(_R, 1)] + [_const_spec(1, 40)] * 3,
    out_specs=[_row_spec(_R, 40)] * 3,
    out_shape=[jax.ShapeDtypeStruct((NP, 40), jnp.float32)] * 3,
)


def kernel(x, adj, W0a, W0b, W0c, b0, W1a, W1b, W1c, b1):
    epad = jnp.full((2, EPAD - N_EDGES), PAD_NODE, jnp.int32)
    adj32 = jnp.concatenate([adj.astype(jnp.int32), epad], axis=1)
    rcw = adj32.reshape(2, NSUB, TSTEPS, CHUNK)
    rc = jnp.stack([rcw[0], rcw[1]], axis=2)
    colr = adj32[1].reshape(NW, DSTEPS, CHUNK)
    xp = jnp.pad(x, ((0, NP - N_NODES), (0, 0)))
    z1 = jnp.zeros((NP,), jnp.float32)

    degp = _sc_degree(colr, z1)
    deg = degp[0, 0] + degp[1, 0] + 1.0
    d = lax.rsqrt(deg)[:, None]
    d2 = (1.0 / deg)[:, None]
    dd = jnp.stack([d, d2], axis=1)

    xs, y0 = _t1(xp, d, W0a)
    s1 = _sc_gather_scatter(128)(xs, rc)
    y1, t1v = _t2(s1, xs, d, d2, W0b)
    s2 = _sc_gather_scatter(128)(t1v, rc)

    ga, gb, gc = _t3(
        s2, t1v, y0, y1, d, W0c,
        b0[None, 0:128], b0[None, 128:256], b0[None, 256:384],
        W1a[0:128], W1a[128:256], W1a[256:384],
        W1b[0:128], W1b[128:256], W1b[256:384],
        W1c[0:128], W1c[128:256], W1c[256:384])
    z24 = jnp.zeros((NP, 24), jnp.float32)
    gbc = jnp.concatenate([gb, z24, gc, z24], axis=1)
    s3 = _sc_gather_scatter(128)(gbc, rc)
    (v,) = _t4(s3.reshape(NCORE, NP, 2, 64),
               gbc.reshape(NP, 2, 64), dd)
    p = v[:, 0, 0:40]
    t2v = jnp.pad(v[:, 1, :], ((0, 0), (0, 64)))
    s4 = _sc_gather_scatter(128)(t2v, rc)
    o0, o1, o2 = _t5(s4, v[:, 1, 0:40], ga, p, d,
                     b1[None, 0:40], b1[None, 40:80], b1[None, 80:120])
    return jnp.concatenate([o0, o1, o2], axis=1)[:N_NODES]
